# Initial kernel scaffold; baseline (speedup 1.0000x reference)
import jax
import jax.numpy as jnp
from jax.experimental import pallas as pl


def kernel(hidden_states, W, b):
    raise NotImplementedError("write your pallas kernel here")


# speedup vs baseline: 5.4533x; 5.4533x over previous
import jax
import jax.numpy as jnp
from jax.experimental import pallas as pl

_HIDDEN = 2048
_EXPERTS = 64
_TOPK = 8
_BLOCK = 512


def _router_kernel(x_ref, wt_ref, b_ref, scores_ref, idx_ref):
    x = x_ref[...]
    logits = jax.lax.dot_general(
        x, wt_ref[...], (((1,), (0,)), ((), ())),
        preferred_element_type=jnp.float32,
    )
    logits = logits + b_ref[...]

    iota = jax.lax.broadcasted_iota(jnp.int32, logits.shape, 1)
    work = logits
    mask = jnp.zeros(logits.shape, jnp.bool_)
    idx_cols = []
    rowmax = None
    for _ in range(_TOPK):
        vmax = jnp.max(work, axis=1, keepdims=True)
        if rowmax is None:
            rowmax = vmax
        is_max = work == vmax
        idx_k = jnp.min(jnp.where(is_max, iota, _EXPERTS), axis=1, keepdims=True)
        sel = iota == idx_k
        mask = jnp.logical_or(mask, sel)
        work = jnp.where(sel, -jnp.inf, work)
        idx_cols.append(idx_k)

    ex = jnp.where(mask, jnp.exp(logits - rowmax), 0.0)
    denom = jnp.sum(ex, axis=1, keepdims=True)
    scores_ref[...] = ex / denom
    idx_ref[...] = jnp.concatenate(idx_cols, axis=1)


def kernel(hidden_states, W, b):
    x = hidden_states.reshape(-1, _HIDDEN)
    n = x.shape[0]
    wt = W.T
    b2 = b.reshape(1, _EXPERTS)
    scores, idx = pl.pallas_call(
        _router_kernel,
        grid=(n // _BLOCK,),
        in_specs=[
            pl.BlockSpec((_BLOCK, _HIDDEN), lambda i: (i, 0)),
            pl.BlockSpec((_HIDDEN, _EXPERTS), lambda i: (0, 0)),
            pl.BlockSpec((1, _EXPERTS), lambda i: (0, 0)),
        ],
        out_specs=[
            pl.BlockSpec((_BLOCK, _EXPERTS), lambda i: (i, 0)),
            pl.BlockSpec((_BLOCK, _TOPK), lambda i: (i, 0)),
        ],
        out_shape=[
            jax.ShapeDtypeStruct((n, _EXPERTS), jnp.float32),
            jax.ShapeDtypeStruct((n, _TOPK), jnp.int32),
        ],
    )(x, wt, b2)
    return (scores, idx)


# speedup vs baseline: 6.5895x; 1.2084x over previous
import jax
import jax.numpy as jnp
from jax.experimental import pallas as pl

_HIDDEN = 2048
_EXPERTS = 64
_TOPK = 8
_BLOCK = 512
_CHUNK = 128


def _router_kernel(x_ref, wt_ref, b_ref, scores_ref, idx_ref):
    x = x_ref[...]
    logits = jax.lax.dot_general(
        x, wt_ref[...], (((1,), (0,)), ((), ())),
        preferred_element_type=jnp.float32,
    )
    logits = logits + b_ref[...]

    for c in range(_BLOCK // _CHUNK):
        work = logits[c * _CHUNK:(c + 1) * _CHUNK, :]
        iota_f = jax.lax.broadcasted_iota(jnp.int32, work.shape, 1).astype(jnp.float32)
        rowmax = None
        idx_cols = []
        weights = []
        sels = []
        for _ in range(_TOPK):
            vmax = jnp.max(work, axis=1, keepdims=True)
            if rowmax is None:
                rowmax = vmax
            is_max = work == vmax
            idx_k = jnp.min(jnp.where(is_max, iota_f, float(_EXPERTS)),
                            axis=1, keepdims=True)
            sel = iota_f == idx_k
            work = jnp.where(sel, -jnp.inf, work)
            idx_cols.append(idx_k)
            weights.append(jnp.exp(vmax - rowmax))
            sels.append(sel)
        denom = weights[0]
        for w in weights[1:]:
            denom = denom + w
        scores = jnp.zeros(work.shape, jnp.float32)
        for k in range(_TOPK):
            scores = jnp.where(sels[k], weights[k] / denom, scores)
        scores_ref[c * _CHUNK:(c + 1) * _CHUNK, :] = scores
        idx_ref[c * _CHUNK:(c + 1) * _CHUNK, :] = (
            jnp.concatenate(idx_cols, axis=1).astype(jnp.int32))


def kernel(hidden_states, W, b):
    x = hidden_states.reshape(-1, _HIDDEN)
    n = x.shape[0]
    wt = W.T
    b2 = b.reshape(1, _EXPERTS)
    scores, idx = pl.pallas_call(
        _router_kernel,
        grid=(n // _BLOCK,),
        in_specs=[
            pl.BlockSpec((_BLOCK, _HIDDEN), lambda i: (i, 0)),
            pl.BlockSpec((_HIDDEN, _EXPERTS), lambda i: (0, 0)),
            pl.BlockSpec((1, _EXPERTS), lambda i: (0, 0)),
        ],
        out_specs=[
            pl.BlockSpec((_BLOCK, _EXPERTS), lambda i: (i, 0)),
            pl.BlockSpec((_BLOCK, _TOPK), lambda i: (i, 0)),
        ],
        out_shape=[
            jax.ShapeDtypeStruct((n, _EXPERTS), jnp.float32),
            jax.ShapeDtypeStruct((n, _TOPK), jnp.int32),
        ],
    )(x, wt, b2)
    return (scores, idx)


# speedup vs baseline: 7.9293x; 1.2033x over previous
import jax
import jax.numpy as jnp
from jax.experimental import pallas as pl

_HIDDEN = 2048
_EXPERTS = 64
_TOPK = 8
_BLOCK = 512
_CHUNK = 128


def _router_kernel(x_ref, wt_ref, b_ref, scores_ref, idx_ref):
    x = x_ref[...]
    logits = jax.lax.dot_general(
        x, wt_ref[...], (((1,), (0,)), ((), ())),
        preferred_element_type=jnp.float32,
    )
    logits = logits + b_ref[...]

    for c in range(_BLOCK // _CHUNK):
        lt = logits[c * _CHUNK:(c + 1) * _CHUNK, :].T
        iota_s = jax.lax.broadcasted_iota(
            jnp.int32, lt.shape, 0).astype(jnp.float32)
        work = lt
        rowmax = None
        denom = None
        idx_rows = []
        for _ in range(_TOPK):
            vmax = jnp.max(work, axis=0, keepdims=True)
            if rowmax is None:
                rowmax = vmax
            is_max = work == vmax
            idx_k = jnp.min(jnp.where(is_max, iota_s, float(_EXPERTS)),
                            axis=0, keepdims=True)
            sel = iota_s == idx_k
            work = jnp.where(sel, -jnp.inf, work)
            idx_rows.append(idx_k)
            w = jnp.exp(vmax - rowmax)
            denom = w if denom is None else denom + w
        recip = 1.0 / denom
        mask = work == -jnp.inf
        scores_t = jnp.where(mask, jnp.exp(lt - rowmax) * recip, 0.0)
        scores_ref[c * _CHUNK:(c + 1) * _CHUNK, :] = scores_t.T
        idx_t = jnp.concatenate(idx_rows, axis=0)
        idx_ref[c * _CHUNK:(c + 1) * _CHUNK, :] = idx_t.T.astype(jnp.int32)


def kernel(hidden_states, W, b):
    x = hidden_states.reshape(-1, _HIDDEN)
    n = x.shape[0]
    wt = W.T
    b2 = b.reshape(1, _EXPERTS)
    scores, idx = pl.pallas_call(
        _router_kernel,
        grid=(n // _BLOCK,),
        in_specs=[
            pl.BlockSpec((_BLOCK, _HIDDEN), lambda i: (i, 0)),
            pl.BlockSpec((_HIDDEN, _EXPERTS), lambda i: (0, 0)),
            pl.BlockSpec((1, _EXPERTS), lambda i: (0, 0)),
        ],
        out_specs=[
            pl.BlockSpec((_BLOCK, _EXPERTS), lambda i: (i, 0)),
            pl.BlockSpec((_BLOCK, _TOPK), lambda i: (i, 0)),
        ],
        out_shape=[
            jax.ShapeDtypeStruct((n, _EXPERTS), jnp.float32),
            jax.ShapeDtypeStruct((n, _TOPK), jnp.int32),
        ],
    )(x, wt, b2)
    return (scores, idx)


# speedup vs baseline: 8.2314x; 1.0381x over previous
import jax
import jax.numpy as jnp
from jax.experimental import pallas as pl

_HIDDEN = 2048
_EXPERTS = 64
_TOPK = 8
_BLOCK = 512
_CHUNK = 128


def _router_kernel(x_ref, wt_ref, b_ref, scores_ref, idx_ref):
    x = x_ref[...]
    logits = jax.lax.dot_general(
        x, wt_ref[...], (((1,), (1,)), ((), ())),
        preferred_element_type=jnp.float32,
    )
    logits = logits + b_ref[...]

    for c in range(_BLOCK // _CHUNK):
        lt = logits[c * _CHUNK:(c + 1) * _CHUNK, :].T
        iota_s = jax.lax.broadcasted_iota(
            jnp.int32, lt.shape, 0).astype(jnp.float32)
        work = lt
        rowmax = None
        denom = None
        idx_rows = []
        for _ in range(_TOPK):
            vmax = jnp.max(work, axis=0, keepdims=True)
            if rowmax is None:
                rowmax = vmax
            is_max = work == vmax
            idx_k = jnp.min(jnp.where(is_max, iota_s, float(_EXPERTS)),
                            axis=0, keepdims=True)
            sel = iota_s == idx_k
            work = jnp.where(sel, -jnp.inf, work)
            idx_rows.append(idx_k)
            w = jnp.exp(vmax - rowmax)
            denom = w if denom is None else denom + w
        recip = 1.0 / denom
        mask = work == -jnp.inf
        scores_t = jnp.where(mask, jnp.exp(lt - rowmax) * recip, 0.0)
        scores_ref[c * _CHUNK:(c + 1) * _CHUNK, :] = scores_t.T
        idx_t = jnp.concatenate(idx_rows, axis=0)
        idx_ref[c * _CHUNK:(c + 1) * _CHUNK, :] = idx_t.T.astype(jnp.int32)


def kernel(hidden_states, W, b):
    x = hidden_states.reshape(-1, _HIDDEN)
    n = x.shape[0]
    b2 = b.reshape(1, _EXPERTS)
    scores, idx = pl.pallas_call(
        _router_kernel,
        grid=(n // _BLOCK,),
        in_specs=[
            pl.BlockSpec((_BLOCK, _HIDDEN), lambda i: (i, 0)),
            pl.BlockSpec((_EXPERTS, _HIDDEN), lambda i: (0, 0)),
            pl.BlockSpec((1, _EXPERTS), lambda i: (0, 0)),
        ],
        out_specs=[
            pl.BlockSpec((_BLOCK, _EXPERTS), lambda i: (i, 0)),
            pl.BlockSpec((_BLOCK, _TOPK), lambda i: (i, 0)),
        ],
        out_shape=[
            jax.ShapeDtypeStruct((n, _EXPERTS), jnp.float32),
            jax.ShapeDtypeStruct((n, _TOPK), jnp.int32),
        ],
    )(x, W, b2)
    return (scores, idx)


# speedup vs baseline: 9.5660x; 1.1621x over previous
import jax
import jax.numpy as jnp
from jax.experimental import pallas as pl

_HIDDEN = 2048
_EXPERTS = 64
_TOPK = 8
_BLOCK = 1024
_CHUNK = 128


def _router_kernel(x_ref, wt_ref, b_ref, scores_ref, idx_ref):
    x = x_ref[...]
    logits = jax.lax.dot_general(
        x, wt_ref[...], (((1,), (1,)), ((), ())),
        preferred_element_type=jnp.float32,
    )
    logits = logits + b_ref[...]

    for c in range(_BLOCK // _CHUNK):
        lt = logits[c * _CHUNK:(c + 1) * _CHUNK, :].T
        iota_s = jax.lax.broadcasted_iota(
            jnp.int32, lt.shape, 0).astype(jnp.float32)
        work = lt
        rowmax = None
        denom = None
        idx_rows = []
        for _ in range(_TOPK):
            vmax = jnp.max(work, axis=0, keepdims=True)
            if rowmax is None:
                rowmax = vmax
            is_max = work == vmax
            idx_k = jnp.min(jnp.where(is_max, iota_s, float(_EXPERTS)),
                            axis=0, keepdims=True)
            sel = iota_s == idx_k
            work = jnp.where(sel, -jnp.inf, work)
            idx_rows.append(idx_k)
            w = jnp.exp(vmax - rowmax)
            denom = w if denom is None else denom + w
        recip = 1.0 / denom
        mask = work == -jnp.inf
        scores_t = jnp.where(mask, jnp.exp(lt - rowmax) * recip, 0.0)
        scores_ref[c * _CHUNK:(c + 1) * _CHUNK, :] = scores_t.T
        idx_t = jnp.concatenate(idx_rows, axis=0)
        idx_ref[c * _CHUNK:(c + 1) * _CHUNK, :] = idx_t.T.astype(jnp.int32)


def kernel(hidden_states, W, b):
    x = hidden_states.reshape(-1, _HIDDEN)
    n = x.shape[0]
    b2 = b.reshape(1, _EXPERTS)
    scores, idx = pl.pallas_call(
        _router_kernel,
        grid=(n // _BLOCK,),
        in_specs=[
            pl.BlockSpec((_BLOCK, _HIDDEN), lambda i: (i, 0)),
            pl.BlockSpec((_EXPERTS, _HIDDEN), lambda i: (0, 0)),
            pl.BlockSpec((1, _EXPERTS), lambda i: (0, 0)),
        ],
        out_specs=[
            pl.BlockSpec((_BLOCK, _EXPERTS), lambda i: (i, 0)),
            pl.BlockSpec((_BLOCK, _TOPK), lambda i: (i, 0)),
        ],
        out_shape=[
            jax.ShapeDtypeStruct((n, _EXPERTS), jnp.float32),
            jax.ShapeDtypeStruct((n, _TOPK), jnp.int32),
        ],
    )(x, W, b2)
    return (scores, idx)


# speedup vs baseline: 9.8189x; 1.0264x over previous
import jax
import jax.numpy as jnp
from jax.experimental import pallas as pl

_HIDDEN = 2048
_EXPERTS = 64
_TOPK = 8
_BLOCK = 2048
_CHUNK = 128


def _router_kernel(x_ref, wt_ref, b_ref, scores_ref, idx_ref):
    x = x_ref[...]
    logits = jax.lax.dot_general(
        x, wt_ref[...], (((1,), (1,)), ((), ())),
        preferred_element_type=jnp.float32,
    )
    logits = logits + b_ref[...]

    for c in range(_BLOCK // _CHUNK):
        lt = logits[c * _CHUNK:(c + 1) * _CHUNK, :].T
        iota_s = jax.lax.broadcasted_iota(
            jnp.int32, lt.shape, 0).astype(jnp.float32)
        work = lt
        rowmax = None
        denom = None
        idx_rows = []
        for _ in range(_TOPK):
            vmax = jnp.max(work, axis=0, keepdims=True)
            if rowmax is None:
                rowmax = vmax
            is_max = work == vmax
            idx_k = jnp.min(jnp.where(is_max, iota_s, float(_EXPERTS)),
                            axis=0, keepdims=True)
            sel = iota_s == idx_k
            work = jnp.where(sel, -jnp.inf, work)
            idx_rows.append(idx_k)
            w = jnp.exp(vmax - rowmax)
            denom = w if denom is None else denom + w
        recip = 1.0 / denom
        mask = work == -jnp.inf
        scores_t = jnp.where(mask, jnp.exp(lt - rowmax) * recip, 0.0)
        scores_ref[c * _CHUNK:(c + 1) * _CHUNK, :] = scores_t.T
        idx_t = jnp.concatenate(idx_rows, axis=0)
        idx_ref[c * _CHUNK:(c + 1) * _CHUNK, :] = idx_t.T.astype(jnp.int32)


def kernel(hidden_states, W, b):
    x = hidden_states.reshape(-1, _HIDDEN)
    n = x.shape[0]
    b2 = b.reshape(1, _EXPERTS)
    scores, idx = pl.pallas_call(
        _router_kernel,
        grid=(n // _BLOCK,),
        in_specs=[
            pl.BlockSpec((_BLOCK, _HIDDEN), lambda i: (i, 0)),
            pl.BlockSpec((_EXPERTS, _HIDDEN), lambda i: (0, 0)),
            pl.BlockSpec((1, _EXPERTS), lambda i: (0, 0)),
        ],
        out_specs=[
            pl.BlockSpec((_BLOCK, _EXPERTS), lambda i: (i, 0)),
            pl.BlockSpec((_BLOCK, _TOPK), lambda i: (i, 0)),
        ],
        out_shape=[
            jax.ShapeDtypeStruct((n, _EXPERTS), jnp.float32),
            jax.ShapeDtypeStruct((n, _TOPK), jnp.int32),
        ],
    )(x, W, b2)
    return (scores, idx)


# speedup vs baseline: 9.8385x; 1.0020x over previous
import jax
import jax.numpy as jnp
from jax.experimental import pallas as pl

_HIDDEN = 2048
_EXPERTS = 64
_TOPK = 8
_BLOCK = 2048
_SPLIT = 2
_SUB = _BLOCK // _SPLIT
_CHUNK = 128


def _router_kernel(*refs):
    x_refs = refs[:_SPLIT]
    wt_ref, b_ref, scores_ref, idx_ref = refs[_SPLIT:]
    for s in range(_SPLIT):
        x = x_refs[s][...]
        logits = jax.lax.dot_general(
            x, wt_ref[...], (((1,), (1,)), ((), ())),
            preferred_element_type=jnp.float32,
        )
        logits = logits + b_ref[...]

        for c in range(_SUB // _CHUNK):
            base = s * _SUB + c * _CHUNK
            lt = logits[c * _CHUNK:(c + 1) * _CHUNK, :].T
            iota_s = jax.lax.broadcasted_iota(
                jnp.int32, lt.shape, 0).astype(jnp.float32)
            work = lt
            rowmax = None
            denom = None
            idx_rows = []
            for _ in range(_TOPK):
                vmax = jnp.max(work, axis=0, keepdims=True)
                if rowmax is None:
                    rowmax = vmax
                is_max = work == vmax
                idx_k = jnp.min(jnp.where(is_max, iota_s, float(_EXPERTS)),
                                axis=0, keepdims=True)
                sel = iota_s == idx_k
                work = jnp.where(sel, -jnp.inf, work)
                idx_rows.append(idx_k)
                w = jnp.exp(vmax - rowmax)
                denom = w if denom is None else denom + w
            recip = 1.0 / denom
            mask = work == -jnp.inf
            scores_t = jnp.where(mask, jnp.exp(lt - rowmax) * recip, 0.0)
            scores_ref[base:base + _CHUNK, :] = scores_t.T
            idx_t = jnp.concatenate(idx_rows, axis=0)
            idx_ref[base:base + _CHUNK, :] = idx_t.T.astype(jnp.int32)


def kernel(hidden_states, W, b):
    x = hidden_states.reshape(-1, _HIDDEN)
    n = x.shape[0]
    b2 = b.reshape(1, _EXPERTS)
    x_specs = [
        pl.BlockSpec((_SUB, _HIDDEN), lambda i, s=s: (i * _SPLIT + s, 0))
        for s in range(_SPLIT)
    ]
    scores, idx = pl.pallas_call(
        _router_kernel,
        grid=(n // _BLOCK,),
        in_specs=x_specs + [
            pl.BlockSpec((_EXPERTS, _HIDDEN), lambda i: (0, 0)),
            pl.BlockSpec((1, _EXPERTS), lambda i: (0, 0)),
        ],
        out_specs=[
            pl.BlockSpec((_BLOCK, _EXPERTS), lambda i: (i, 0)),
            pl.BlockSpec((_BLOCK, _TOPK), lambda i: (i, 0)),
        ],
        out_shape=[
            jax.ShapeDtypeStruct((n, _EXPERTS), jnp.float32),
            jax.ShapeDtypeStruct((n, _TOPK), jnp.int32),
        ],
    )(*([x] * _SPLIT), W, b2)
    return (scores, idx)


# speedup vs baseline: 9.8385x; 1.0000x over previous
import jax
import jax.numpy as jnp
from jax.experimental import pallas as pl

_HIDDEN = 2048
_EXPERTS = 64
_TOPK = 8
_BLOCK = 2048
_SPLIT = 4
_SUB = _BLOCK // _SPLIT
_CHUNK = 128


def _router_kernel(*refs):
    x_refs = refs[:_SPLIT]
    wt_ref, b_ref, scores_ref, idx_ref = refs[_SPLIT:]
    for s in range(_SPLIT):
        x = x_refs[s][...]
        logits = jax.lax.dot_general(
            x, wt_ref[...], (((1,), (1,)), ((), ())),
            preferred_element_type=jnp.float32,
        )
        logits = logits + b_ref[...]

        for c in range(_SUB // _CHUNK):
            base = s * _SUB + c * _CHUNK
            lt = logits[c * _CHUNK:(c + 1) * _CHUNK, :].T
            iota_s = jax.lax.broadcasted_iota(
                jnp.int32, lt.shape, 0).astype(jnp.float32)
            work = lt
            rowmax = None
            denom = None
            idx_rows = []
            for _ in range(_TOPK):
                vmax = jnp.max(work, axis=0, keepdims=True)
                if rowmax is None:
                    rowmax = vmax
                is_max = work == vmax
                idx_k = jnp.min(jnp.where(is_max, iota_s, float(_EXPERTS)),
                                axis=0, keepdims=True)
                sel = iota_s == idx_k
                work = jnp.where(sel, -jnp.inf, work)
                idx_rows.append(idx_k)
                w = jnp.exp(vmax - rowmax)
                denom = w if denom is None else denom + w
            recip = 1.0 / denom
            mask = work == -jnp.inf
            scores_t = jnp.where(mask, jnp.exp(lt - rowmax) * recip, 0.0)
            scores_ref[base:base + _CHUNK, :] = scores_t.T
            idx_t = jnp.concatenate(idx_rows, axis=0)
            idx_ref[base:base + _CHUNK, :] = idx_t.T.astype(jnp.int32)


def kernel(hidden_states, W, b):
    x = hidden_states.reshape(-1, _HIDDEN)
    n = x.shape[0]
    b2 = b.reshape(1, _EXPERTS)
    x_specs = [
        pl.BlockSpec((_SUB, _HIDDEN), lambda i, s=s: (i * _SPLIT + s, 0))
        for s in range(_SPLIT)
    ]
    scores, idx = pl.pallas_call(
        _router_kernel,
        grid=(n // _BLOCK,),
        in_specs=x_specs + [
            pl.BlockSpec((_EXPERTS, _HIDDEN), lambda i: (0, 0)),
            pl.BlockSpec((1, _EXPERTS), lambda i: (0, 0)),
        ],
        out_specs=[
            pl.BlockSpec((_BLOCK, _EXPERTS), lambda i: (i, 0)),
            pl.BlockSpec((_BLOCK, _TOPK), lambda i: (i, 0)),
        ],
        out_shape=[
            jax.ShapeDtypeStruct((n, _EXPERTS), jnp.float32),
            jax.ShapeDtypeStruct((n, _TOPK), jnp.int32),
        ],
    )(*([x] * _SPLIT), W, b2)
    return (scores, idx)
